# Initial kernel scaffold; baseline (speedup 1.0000x reference)
import jax
import jax.numpy as jnp
from jax.experimental import pallas as pl


def kernel(data, emb_tables, W0, b0, W1, b1, W2, b2, W3, b3):
    raise NotImplementedError("write your pallas kernel here")


# speedup vs baseline: 2.0180x; 2.0180x over previous
import functools

import jax
import jax.numpy as jnp
from jax import lax
from jax.experimental import pallas as pl
from jax.experimental.pallas import tpu as pltpu
from jax.experimental.pallas import tpu_sc as plsc

N_CAT = 26
N_NUM = 13
VOCAB = 100000
EMB_DIM = 32
BATCH = 16384
IN_DIM = N_CAT * EMB_DIM

NW = 32
C = 128
ROWS = BATCH * N_CAT
ROWS_PER_W = ROWS // NW
NCHUNK = ROWS_PER_W // C
NBUF = 4
NGROUP = NCHUNK // NBUF


def _sc_gather_body(table_hbm, gidx_hbm, out_hbm, idx_v, rows_v,
                    s0, s1, s2, s3):
    sems = (s0, s1, s2, s3)
    wid = lax.axis_index("s") * 2 + lax.axis_index("c")
    base = wid * ROWS_PER_W
    pltpu.sync_copy(gidx_hbm.at[wid], idx_v)

    for b in range(NBUF):
        pltpu.async_copy(table_hbm.at[idx_v.at[b]], rows_v.at[b], sems[b])

    def group(g, carry):
        for b in range(NBUF):
            j = g * NBUF + b
            pltpu.make_async_copy(
                table_hbm.at[idx_v.at[j]], rows_v.at[b], sems[b]).wait()
            pltpu.sync_copy(rows_v.at[b],
                            out_hbm.at[pl.ds(base + j * C, C)])

            @pl.when(j + NBUF < NCHUNK)
            def _():
                pltpu.async_copy(table_hbm.at[idx_v.at[j + NBUF]],
                                 rows_v.at[b], sems[b])
        return carry

    lax.fori_loop(0, NGROUP, group, 0)


@jax.jit
def _sc_gather(table, gidx):
    mesh = plsc.VectorSubcoreMesh(core_axis_name="c", subcore_axis_name="s")
    f = functools.partial(
        pl.kernel,
        out_type=jax.ShapeDtypeStruct((ROWS, EMB_DIM), jnp.float32),
        mesh=mesh,
        scratch_types=[
            pltpu.VMEM((NCHUNK, C), jnp.int32),
            pltpu.VMEM((NBUF, C, EMB_DIM), jnp.float32),
            pltpu.SemaphoreType.DMA,
            pltpu.SemaphoreType.DMA,
            pltpu.SemaphoreType.DMA,
            pltpu.SemaphoreType.DMA,
        ],
        compiler_params=pltpu.CompilerParams(use_tc_tiling_on_sc=False),
    )(_sc_gather_body)
    return f(table, gidx)


BM = 1024


def _mlp_body(emb_ref, num_ref, w0e_ref, w0n_ref, b0_ref, w1_ref, b1_ref,
              w2_ref, b2_ref, w3_ref, b3_ref, out_ref):
    x = jnp.dot(emb_ref[...], w0e_ref[...], preferred_element_type=jnp.float32)
    x = x + jnp.dot(num_ref[...], w0n_ref[...],
                    preferred_element_type=jnp.float32)
    x = jnp.maximum(x + b0_ref[...], 0.0)
    x = jnp.maximum(
        jnp.dot(x, w1_ref[...], preferred_element_type=jnp.float32)
        + b1_ref[...], 0.0)
    x = jnp.maximum(
        jnp.dot(x, w2_ref[...], preferred_element_type=jnp.float32)
        + b2_ref[...], 0.0)
    logits = jnp.maximum(
        jnp.dot(x, w3_ref[...], preferred_element_type=jnp.float32)
        + b3_ref[...], 0.0)
    l0 = logits[:, 0:1]
    loc = logits[:, 1:2]
    l2 = logits[:, 2:3]
    p = 1.0 / (1.0 + jnp.exp(-l0))
    scale = jnp.maximum(l2, 0.0) + jnp.log1p(jnp.exp(-jnp.abs(l2)))
    out_ref[...] = p * jnp.exp(loc + 0.5 * scale * scale)


@jax.jit
def _mlp(emb, num, w0e, w0n, b0, w1, b1, w2, b2, w3, b3):
    full = lambda i: (0, 0)
    return pl.pallas_call(
        _mlp_body,
        grid=(BATCH // BM,),
        in_specs=[
            pl.BlockSpec((BM, IN_DIM), lambda i: (i, 0)),
            pl.BlockSpec((BM, N_NUM), lambda i: (i, 0)),
            pl.BlockSpec(w0e.shape, full),
            pl.BlockSpec(w0n.shape, full),
            pl.BlockSpec(b0.shape, full),
            pl.BlockSpec(w1.shape, full),
            pl.BlockSpec(b1.shape, full),
            pl.BlockSpec(w2.shape, full),
            pl.BlockSpec(b2.shape, full),
            pl.BlockSpec(w3.shape, full),
            pl.BlockSpec(b3.shape, full),
        ],
        out_specs=pl.BlockSpec((BM, 1), lambda i: (i, 0)),
        out_shape=jax.ShapeDtypeStruct((BATCH, 1), jnp.float32),
    )(emb, num, w0e, w0n, b0, w1, b1, w2, b2, w3, b3)


def kernel(data, emb_tables, W0, b0, W1, b1, W2, b2, W3, b3):
    cat = data[:, :N_CAT].astype(jnp.int32)
    gidx = (cat + jnp.arange(N_CAT, dtype=jnp.int32) * VOCAB)
    gidx = gidx.reshape(NW, NCHUNK, C)
    table = emb_tables.reshape(N_CAT * VOCAB, EMB_DIM)
    emb = _sc_gather(table, gidx).reshape(BATCH, IN_DIM)
    num = data[:, N_CAT:]
    return _mlp(emb, num,
                W0[:IN_DIM], W0[IN_DIM:], b0.reshape(1, -1),
                W1, b1.reshape(1, -1), W2, b2.reshape(1, -1),
                W3, b3.reshape(1, -1))
